# baseline (speedup 1.0000x reference)
import functools
import jax
import jax.numpy as jnp
from jax import lax
from jax.experimental import pallas as pl
from jax.experimental.pallas import tpu as pltpu
from jax.experimental.pallas import tpu_sc as plsc

NC, NS, L = 2, 16, 16
NW = NC * NS
K = 64


def _sc_sample(T, x0, x1, wx0, wx1, y0g, y1g, wy0, wy1, B, H, W, C):
    P = H * W
    wids_per_b = NW // B
    i_per_wid = H // wids_per_b
    chunks_per_row = W // K
    n_chunks = i_per_wid * chunks_per_row

    mesh = plsc.VectorSubcoreMesh(core_axis_name="c", subcore_axis_name="s")

    @functools.partial(
        pl.kernel,
        mesh=mesh,
        out_type=jax.ShapeDtypeStruct((B * P, C), jnp.float32),
        compiler_params=pltpu.CompilerParams(
            needs_layout_passes=False, use_tc_tiling_on_sc=False
        ),
        scratch_types=[
            pltpu.VMEM((W,), jnp.int32),
            pltpu.VMEM((W,), jnp.int32),
            pltpu.VMEM((W,), jnp.float32),
            pltpu.VMEM((W,), jnp.float32),
            pltpu.VMEM((H,), jnp.int32),
            pltpu.VMEM((H,), jnp.int32),
            pltpu.VMEM((H,), jnp.float32),
            pltpu.VMEM((H,), jnp.float32),
            pltpu.VMEM((K,), jnp.int32),
            pltpu.VMEM((K,), jnp.int32),
            pltpu.VMEM((K,), jnp.int32),
            pltpu.VMEM((K,), jnp.int32),
            pltpu.VMEM((K, C), jnp.float32),
            pltpu.VMEM((K, C), jnp.float32),
            pltpu.VMEM((K, C), jnp.float32),
            pltpu.VMEM((K, C), jnp.float32),
            pltpu.VMEM((K, C), jnp.float32),
            pltpu.SemaphoreType.DMA,
            pltpu.SemaphoreType.DMA,
        ],
    )
    def k(T_hbm, x0_hbm, x1_hbm, wx0_hbm, wx1_hbm, y0_hbm, y1_hbm,
          wy0_hbm, wy1_hbm, out_hbm,
          x0v, x1v, wx0v, wx1v, y0v, y1v, wy0v, wy1v,
          idxa, idxb, idxc, idxd, Ga, Gb, Gc, Gd, Ob, gsem, osem):
        wid = lax.axis_index("s") * NC + lax.axis_index("c")
        b = wid // wids_per_b
        i_base = (wid % wids_per_b) * i_per_wid

        pltpu.sync_copy(x0_hbm.at[b], x0v)
        pltpu.sync_copy(x1_hbm.at[b], x1v)
        pltpu.sync_copy(wx0_hbm.at[b], wx0v)
        pltpu.sync_copy(wx1_hbm.at[b], wx1v)
        pltpu.sync_copy(y0_hbm.at[b], y0v)
        pltpu.sync_copy(y1_hbm.at[b], y1v)
        pltpu.sync_copy(wy0_hbm.at[b], wy0v)
        pltpu.sync_copy(wy1_hbm.at[b], wy1v)

        iota = lax.iota(jnp.int32, L)

        def chunk(t, carry):
            i_loc = t // chunks_per_row
            j0 = (t % chunks_per_row) * K
            i_abs = i_base + i_loc
            i_splat = jnp.full((L,), i_abs, dtype=jnp.int32)
            ya = plsc.load_gather(y0v, [i_splat])
            yb = plsc.load_gather(y1v, [i_splat])
            for g in range(K // L):
                j16 = jnp.full((L,), j0 + g * L, dtype=jnp.int32) + iota
                xa = plsc.load_gather(x0v, [j16])
                xb = plsc.load_gather(x1v, [j16])
                idxa[pl.ds(g * L, L)] = ya + xa
                idxb[pl.ds(g * L, L)] = yb + xa
                idxc[pl.ds(g * L, L)] = ya + xb
                idxd[pl.ds(g * L, L)] = yb + xb
            da = pltpu.async_copy(T_hbm.at[idxa], Ga, gsem)
            db = pltpu.async_copy(T_hbm.at[idxb], Gb, gsem)
            dc = pltpu.async_copy(T_hbm.at[idxc], Gc, gsem)
            dd = pltpu.async_copy(T_hbm.at[idxd], Gd, gsem)
            da.wait()
            db.wait()
            dc.wait()
            dd.wait()

            gy0 = plsc.load_gather(wy0v, [i_splat])
            gy1 = plsc.load_gather(wy1v, [i_splat])
            for g in range(K // L):
                j16 = jnp.full((L,), j0 + g * L, dtype=jnp.int32) + iota
                gx0 = plsc.load_gather(wx0v, [j16])
                gx1 = plsc.load_gather(wx1v, [j16])
                wa = gy0 * gx0
                wb = gy1 * gx0
                wc = gy0 * gx1
                wd = gy1 * gx1
                r16 = jnp.full((L,), g * L, dtype=jnp.int32) + iota

                def cstep(c, carry2):
                    c_splat = jnp.full((L,), c, dtype=jnp.int32)
                    va = plsc.load_gather(Ga, [r16, c_splat])
                    vb = plsc.load_gather(Gb, [r16, c_splat])
                    vc = plsc.load_gather(Gc, [r16, c_splat])
                    vd = plsc.load_gather(Gd, [r16, c_splat])
                    acc = wa * va + wb * vb + wc * vc + wd * vd
                    plsc.store_scatter(Ob, [r16, c_splat], acc)
                    return carry2

                lax.fori_loop(0, C, cstep, 0, unroll=8)

            row_base = b * P + i_abs * W + j0
            pltpu.async_copy(Ob, out_hbm.at[pl.ds(row_base, K)], osem).wait()
            return carry

        lax.fori_loop(0, n_chunks, chunk, 0)

    return k(T, x0, x1, wx0, wx1, y0g, y1g, wy0, wy1)


def kernel(U, theta, out_size):
    B, C, H, W = U.shape
    oh, ow = H, W
    zero = (jnp.asarray(out_size) - oh).astype(U.dtype)
    ox = jnp.linspace(-1.0, 1.0, ow)
    oy = jnp.linspace(-1.0, 1.0, oh)
    x = (theta[:, 0, 0][:, None] + ox[None, :]) + zero
    y = (theta[:, 1, 0][:, None] + oy[None, :]) + zero
    x = (x + 1.0) * (float(W) - 1.0) / 2.0
    y = (y + 1.0) * (float(H) - 1.0) / 2.0
    x0 = jnp.clip(jnp.floor(x).astype(jnp.int32), 0, W - 2)
    x1 = jnp.clip(jnp.ceil(x).astype(jnp.int32), 0, W - 1)
    y0 = jnp.clip(jnp.floor(y).astype(jnp.int32), 0, H - 2)
    y1 = jnp.clip(jnp.ceil(y).astype(jnp.int32), 0, H - 1)
    wx0 = x1.astype(x.dtype) - x
    wx1 = x - x0.astype(x.dtype)
    wy0 = y1.astype(y.dtype) - y
    wy1 = y - y0.astype(y.dtype)
    bbase = (jnp.arange(B, dtype=jnp.int32) * (H * W))[:, None]
    y0g = bbase + y0 * W
    y1g = bbase + y1 * W

    T = U.reshape(B * H * W, C)
    out = _sc_sample(T, x0, x1, wx0, wx1, y0g, y1g, wy0, wy1, B, H, W, C)
    return out.reshape(B, C, oh, ow)


# speedup vs baseline: 1.1620x; 1.1620x over previous
import functools
import jax
import jax.numpy as jnp
from jax import lax
from jax.experimental import pallas as pl
from jax.experimental.pallas import tpu as pltpu
from jax.experimental.pallas import tpu_sc as plsc

NC, NS, L = 2, 16, 16
NW = NC * NS
K = 96


def _sc_sample(T, x0, x1, wx0, wx1, y0g, y1g, wy0, wy1, B, H, W, C):
    P = H * W
    wids_per_b = NW // B
    i_per_wid = H // wids_per_b
    chunks_per_row = W // K
    n_chunks = i_per_wid * chunks_per_row

    mesh = plsc.VectorSubcoreMesh(core_axis_name="c", subcore_axis_name="s")

    @functools.partial(
        pl.kernel,
        mesh=mesh,
        out_type=jax.ShapeDtypeStruct((B * P, C), jnp.float32),
        compiler_params=pltpu.CompilerParams(
            needs_layout_passes=False, use_tc_tiling_on_sc=False
        ),
        scratch_types=[
            pltpu.VMEM((W,), jnp.int32),
            pltpu.VMEM((W,), jnp.int32),
            pltpu.VMEM((W,), jnp.float32),
            pltpu.VMEM((W,), jnp.float32),
            pltpu.VMEM((H,), jnp.int32),
            pltpu.VMEM((H,), jnp.int32),
            pltpu.VMEM((H,), jnp.float32),
            pltpu.VMEM((H,), jnp.float32),
            pltpu.VMEM((2, K), jnp.int32),
            pltpu.VMEM((2, K), jnp.int32),
            pltpu.VMEM((2, K), jnp.int32),
            pltpu.VMEM((2, K), jnp.int32),
            pltpu.VMEM((2, K, C), jnp.float32),
            pltpu.VMEM((2, K, C), jnp.float32),
            pltpu.VMEM((2, K, C), jnp.float32),
            pltpu.VMEM((2, K, C), jnp.float32),
            pltpu.VMEM((2, K, C), jnp.float32),
            pltpu.SemaphoreType.DMA,
            pltpu.SemaphoreType.DMA,
        ],
    )
    def k(T_hbm, x0_hbm, x1_hbm, wx0_hbm, wx1_hbm, y0_hbm, y1_hbm,
          wy0_hbm, wy1_hbm, out_hbm,
          x0v, x1v, wx0v, wx1v, y0v, y1v, wy0v, wy1v,
          idxa, idxb, idxc, idxd, Ga, Gb, Gc, Gd, Ob, gsem, osem):
        wid = lax.axis_index("s") * NC + lax.axis_index("c")
        b = wid // wids_per_b
        i_base = (wid % wids_per_b) * i_per_wid

        pltpu.sync_copy(x0_hbm.at[b], x0v)
        pltpu.sync_copy(x1_hbm.at[b], x1v)
        pltpu.sync_copy(wx0_hbm.at[b], wx0v)
        pltpu.sync_copy(wx1_hbm.at[b], wx1v)
        pltpu.sync_copy(y0_hbm.at[b], y0v)
        pltpu.sync_copy(y1_hbm.at[b], y1v)
        pltpu.sync_copy(wy0_hbm.at[b], wy0v)
        pltpu.sync_copy(wy1_hbm.at[b], wy1v)

        iota = lax.iota(jnp.int32, L)
        G = (Ga, Gb, Gc, Gd)
        IDX = (idxa, idxb, idxc, idxd)

        def build_and_fire(t, p):
            i_loc = t // chunks_per_row
            j0 = (t % chunks_per_row) * K
            i_splat = jnp.full((L,), i_base + i_loc, dtype=jnp.int32)
            ya = plsc.load_gather(y0v, [i_splat])
            yb = plsc.load_gather(y1v, [i_splat])
            for g in range(K // L):
                j16 = jnp.full((L,), j0 + g * L, dtype=jnp.int32) + iota
                xa = plsc.load_gather(x0v, [j16])
                xb = plsc.load_gather(x1v, [j16])
                idxa[p, pl.ds(g * L, L)] = ya + xa
                idxb[p, pl.ds(g * L, L)] = yb + xa
                idxc[p, pl.ds(g * L, L)] = ya + xb
                idxd[p, pl.ds(g * L, L)] = yb + xb
            for q in range(4):
                pltpu.async_copy(T_hbm.at[IDX[q].at[p]], G[q].at[p], gsem)

        def wait_gathers(p):
            for q in range(4):
                pltpu.make_async_copy(T_hbm.at[IDX[q].at[p]], G[q].at[p],
                                      gsem).wait()

        def combine_and_out(t, p):
            i_loc = t // chunks_per_row
            j0 = (t % chunks_per_row) * K
            i_splat = jnp.full((L,), i_base + i_loc, dtype=jnp.int32)
            gy0 = plsc.load_gather(wy0v, [i_splat])
            gy1 = plsc.load_gather(wy1v, [i_splat])
            for g in range(K // L):
                j16 = jnp.full((L,), j0 + g * L, dtype=jnp.int32) + iota
                gx0 = plsc.load_gather(wx0v, [j16])
                gx1 = plsc.load_gather(wx1v, [j16])
                wa = gy0 * gx0
                wb = gy1 * gx0
                wc = gy0 * gx1
                wd = gy1 * gx1
                r16 = jnp.full((L,), g * L, dtype=jnp.int32) + iota

                def cstep(c, carry2):
                    c_splat = jnp.full((L,), c, dtype=jnp.int32)
                    va = plsc.load_gather(Ga.at[p], [r16, c_splat])
                    vb = plsc.load_gather(Gb.at[p], [r16, c_splat])
                    vc = plsc.load_gather(Gc.at[p], [r16, c_splat])
                    vd = plsc.load_gather(Gd.at[p], [r16, c_splat])
                    acc = wa * va + wb * vb + wc * vc + wd * vd
                    plsc.store_scatter(Ob.at[p], [r16, c_splat], acc)
                    return carry2

                lax.fori_loop(0, C, cstep, 0, unroll=8)

            row_base = b * P + (i_base + i_loc) * W + j0
            pltpu.async_copy(Ob.at[p], out_hbm.at[pl.ds(row_base, K)], osem)

        def drain_one_out(p):
            pltpu.make_async_copy(Ob.at[p], out_hbm.at[pl.ds(0, K)],
                                  osem).wait()

        build_and_fire(0, 0)

        def pair(u, carry):
            build_and_fire(2 * u + 1, 1)
            wait_gathers(0)

            @pl.when(u > 0)
            def _():
                drain_one_out(0)

            combine_and_out(2 * u, 0)

            @pl.when(2 * u + 2 < n_chunks)
            def _():
                build_and_fire(2 * u + 2, 0)

            wait_gathers(1)

            @pl.when(u > 0)
            def _():
                drain_one_out(1)

            combine_and_out(2 * u + 1, 1)
            return carry

        lax.fori_loop(0, n_chunks // 2, pair, 0)
        drain_one_out(0)
        drain_one_out(1)

    return k(T, x0, x1, wx0, wx1, y0g, y1g, wy0, wy1)


def kernel(U, theta, out_size):
    B, C, H, W = U.shape
    oh, ow = H, W
    zero = (jnp.asarray(out_size) - oh).astype(U.dtype)
    ox = jnp.linspace(-1.0, 1.0, ow)
    oy = jnp.linspace(-1.0, 1.0, oh)
    x = (theta[:, 0, 0][:, None] + ox[None, :]) + zero
    y = (theta[:, 1, 0][:, None] + oy[None, :]) + zero
    x = (x + 1.0) * (float(W) - 1.0) / 2.0
    y = (y + 1.0) * (float(H) - 1.0) / 2.0
    x0 = jnp.clip(jnp.floor(x).astype(jnp.int32), 0, W - 2)
    x1 = jnp.clip(jnp.ceil(x).astype(jnp.int32), 0, W - 1)
    y0 = jnp.clip(jnp.floor(y).astype(jnp.int32), 0, H - 2)
    y1 = jnp.clip(jnp.ceil(y).astype(jnp.int32), 0, H - 1)
    wx0 = x1.astype(x.dtype) - x
    wx1 = x - x0.astype(x.dtype)
    wy0 = y1.astype(y.dtype) - y
    wy1 = y - y0.astype(y.dtype)
    bbase = (jnp.arange(B, dtype=jnp.int32) * (H * W))[:, None]
    y0g = bbase + y0 * W
    y1g = bbase + y1 * W

    T = U.reshape(B * H * W, C)
    out = _sc_sample(T, x0, x1, wx0, wx1, y0g, y1g, wy0, wy1, B, H, W, C)
    return out.reshape(B, C, oh, ow)


# speedup vs baseline: 4.6113x; 3.9684x over previous
import functools
import jax
import jax.numpy as jnp
from jax import lax
from jax.experimental import pallas as pl
from jax.experimental.pallas import tpu as pltpu
from jax.experimental.pallas import tpu_sc as plsc

NC, NS, L = 2, 16, 16
NW = NC * NS
NGRP = 16
NHALF = 2
NB = 4
PF = 2


def _sc_sample(T2, scal, xm0, xm1, wxa0, wxa1, yr0, yr1, wya0, wya1,
               B, H, W, C):
    P = H * W
    NI = H // NGRP
    JW = W // NHALF
    WW = JW + 2
    RWORDS = WW * C
    OWORDS = JW * C
    JPAD = xm0.shape[-1]
    IPAD = yr0.shape[-1]

    mesh = plsc.VectorSubcoreMesh(core_axis_name="c", subcore_axis_name="s")

    @functools.partial(
        pl.kernel,
        mesh=mesh,
        out_type=jax.ShapeDtypeStruct((B * P * C,), jnp.float32),
        compiler_params=pltpu.CompilerParams(
            needs_layout_passes=False, use_tc_tiling_on_sc=False
        ),
        scratch_types=[
            pltpu.VMEM((16,), jnp.int32),
            pltpu.VMEM((JPAD,), jnp.int32),
            pltpu.VMEM((JPAD,), jnp.int32),
            pltpu.VMEM((JPAD,), jnp.float32),
            pltpu.VMEM((JPAD,), jnp.float32),
            pltpu.VMEM((IPAD,), jnp.int32),
            pltpu.VMEM((IPAD,), jnp.int32),
            pltpu.VMEM((IPAD,), jnp.float32),
            pltpu.VMEM((IPAD,), jnp.float32),
            pltpu.VMEM((NB, RWORDS), jnp.float32),
            pltpu.VMEM((2, OWORDS), jnp.float32),
            pltpu.SemaphoreType.DMA,
            pltpu.SemaphoreType.DMA,
        ],
    )
    def k(T2_hbm, scal_hbm, xm0_hbm, xm1_hbm, wx0_hbm, wx1_hbm,
          yr0_hbm, yr1_hbm, wy0_hbm, wy1_hbm, out_hbm,
          scal_v, xm0v, xm1v, wx0v, wx1v, yr0v, yr1v, wy0v, wy1v,
          RB, OB, rsem, osem):
        wid = lax.axis_index("s") * NC + lax.axis_index("c")
        g = wid // NHALF
        h = wid % NHALF
        i0 = g * NI
        j0 = h * JW

        def fire_row(b, sy, tx, r):
            src = T2_hbm.at[b * H + sy + r, pl.ds(tx * C, RWORDS)]
            pltpu.async_copy(src, RB.at[r % NB], rsem)

        def wait_row(b):
            pltpu.make_async_copy(
                T2_hbm.at[b * H, pl.ds(0, RWORDS)], RB.at[0], rsem
            ).wait()

        def fire_out(b, i_abs, p):
            dst = out_hbm.at[pl.ds((b * P + i_abs * W + j0) * C, OWORDS)]
            pltpu.async_copy(OB.at[p], dst, osem)

        def drain_out():
            pltpu.make_async_copy(
                OB.at[0], out_hbm.at[pl.ds(0, OWORDS)], osem
            ).wait()

        def batch(b, carry0):
            pltpu.sync_copy(scal_hbm.at[b, wid], scal_v)
            pltpu.sync_copy(xm0_hbm.at[b, h], xm0v)
            pltpu.sync_copy(xm1_hbm.at[b, h], xm1v)
            pltpu.sync_copy(wx0_hbm.at[b, h], wx0v)
            pltpu.sync_copy(wx1_hbm.at[b, h], wx1v)
            pltpu.sync_copy(yr0_hbm.at[b, g], yr0v)
            pltpu.sync_copy(yr1_hbm.at[b, g], yr1v)
            pltpu.sync_copy(wy0_hbm.at[b, g], wy0v)
            pltpu.sync_copy(wy1_hbm.at[b, g], wy1v)
            sv = scal_v[pl.ds(0, L)]
            tx = sv[0]
            sy = sv[1]
            lastv = yr1v[pl.ds(NI - 1, L)]
            cap = lastv[0] + 1

            def row(i_loc, carry):
                fired, waited = carry
                yv0 = yr0v[pl.ds(i_loc, L)]
                yv1 = yr1v[pl.ds(i_loc, L)]
                o0 = yv0[0]
                o1 = yv1[0]
                need = o1 + 1
                want = jnp.minimum(need + PF, cap)

                def fcond(s):
                    return s < want

                def fbody(s):
                    fire_row(b, sy, tx, s)
                    return s + 1

                fired = lax.while_loop(fcond, fbody, fired)

                def wcond(s):
                    return s < need

                def wbody(s):
                    wait_row(b)
                    return s + 1

                waited = lax.while_loop(wcond, wbody, waited)

                s0 = o0 % NB
                s1 = o1 % NB
                wv0 = wy0v[pl.ds(i_loc, L)]
                wv1 = wy1v[pl.ds(i_loc, L)]
                gy0 = jnp.full((L,), wv0[0], dtype=jnp.float32)
                gy1 = jnp.full((L,), wv1[0], dtype=jnp.float32)
                p = i_loc % 2

                t_glob = b * NI + i_loc

                @pl.when(t_glob >= 2)
                def _():
                    drain_out()

                def jblock(jb, c2):
                    xb0 = xm0v[pl.ds(jb * L, L)]
                    xb1 = xm1v[pl.ds(jb * L, L)]
                    wb0 = wx0v[pl.ds(jb * L, L)]
                    wb1 = wx1v[pl.ds(jb * L, L)]
                    for kk in range(L):
                        m0 = xb0[kk]
                        m1 = xb1[kk]
                        wA = gy0 * jnp.full((L,), wb0[kk], dtype=jnp.float32)
                        wC = gy0 * jnp.full((L,), wb1[kk], dtype=jnp.float32)
                        wB = gy1 * jnp.full((L,), wb0[kk], dtype=jnp.float32)
                        wD = gy1 * jnp.full((L,), wb1[kk], dtype=jnp.float32)
                        ob = (jb * L + kk) * C
                        for cc in range(C // L):
                            va = RB[s0, pl.ds(m0 + cc * L, L)]
                            vb = RB[s1, pl.ds(m0 + cc * L, L)]
                            vc = RB[s0, pl.ds(m1 + cc * L, L)]
                            vd = RB[s1, pl.ds(m1 + cc * L, L)]
                            acc = wA * va + wB * vb + wC * vc + wD * vd
                            OB[p, pl.ds(ob + cc * L, L)] = acc
                    return c2

                lax.fori_loop(0, JW // L, jblock, 0)
                fire_out(b, i0 + i_loc, p)
                return (fired, waited)

            lax.fori_loop(0, NI, row, (jnp.int32(0), jnp.int32(0)))
            return carry0

        lax.fori_loop(0, B, batch, 0)
        drain_out()
        drain_out()

    return k(T2, scal, xm0, xm1, wxa0, wxa1, yr0, yr1, wya0, wya1)


def kernel(U, theta, out_size):
    B, C, H, W = U.shape
    oh, ow = H, W
    P = H * W
    NI = H // NGRP
    JW = W // NHALF
    WW = JW + 2
    NR = NI + 2
    zero = (jnp.asarray(out_size) - oh).astype(U.dtype)
    ox = jnp.linspace(-1.0, 1.0, ow)
    oy = jnp.linspace(-1.0, 1.0, oh)
    x = (theta[:, 0, 0][:, None] + ox[None, :]) + zero
    y = (theta[:, 1, 0][:, None] + oy[None, :]) + zero
    x = (x + 1.0) * (float(W) - 1.0) / 2.0
    y = (y + 1.0) * (float(H) - 1.0) / 2.0
    x0 = jnp.clip(jnp.floor(x).astype(jnp.int32), 0, W - 2)
    x1 = jnp.clip(jnp.ceil(x).astype(jnp.int32), 0, W - 1)
    y0 = jnp.clip(jnp.floor(y).astype(jnp.int32), 0, H - 2)
    y1 = jnp.clip(jnp.ceil(y).astype(jnp.int32), 0, H - 1)
    wx0 = x1.astype(x.dtype) - x
    wx1 = x - x0.astype(x.dtype)
    wy0 = y1.astype(y.dtype) - y
    wy1 = y - y0.astype(y.dtype)

    tx = jnp.minimum(x0[:, ::JW], W - WW)
    sy = jnp.minimum(y0[:, ::NI], H - NR)
    gg = jnp.arange(NW, dtype=jnp.int32) // NHALF
    hh = jnp.arange(NW, dtype=jnp.int32) % NHALF
    scal = jnp.zeros((B, NW, 16), jnp.int32)
    scal = scal.at[:, :, 0].set(tx[:, hh])
    scal = scal.at[:, :, 1].set(sy[:, gg])

    JPAD = JW + 16
    xr = x0.reshape(B, NHALF, JW) - tx[:, :, None]
    xs = x1.reshape(B, NHALF, JW) - tx[:, :, None]
    xm0 = jnp.zeros((B, NHALF, JPAD), jnp.int32).at[:, :, :JW].set(xr * C)
    xm1 = jnp.zeros((B, NHALF, JPAD), jnp.int32).at[:, :, :JW].set(xs * C)
    wxa0 = jnp.zeros((B, NHALF, JPAD), jnp.float32).at[:, :, :JW].set(
        wx0.reshape(B, NHALF, JW))
    wxa1 = jnp.zeros((B, NHALF, JPAD), jnp.float32).at[:, :, :JW].set(
        wx1.reshape(B, NHALF, JW))

    IPAD = NI + 24
    yrr0 = y0.reshape(B, NGRP, NI) - sy[:, :, None]
    yrr1 = y1.reshape(B, NGRP, NI) - sy[:, :, None]
    yr0 = jnp.zeros((B, NGRP, IPAD), jnp.int32).at[:, :, :NI].set(yrr0)
    yr1 = jnp.zeros((B, NGRP, IPAD), jnp.int32).at[:, :, :NI].set(yrr1)
    wya0 = jnp.zeros((B, NGRP, IPAD), jnp.float32).at[:, :, :NI].set(
        wy0.reshape(B, NGRP, NI))
    wya1 = jnp.zeros((B, NGRP, IPAD), jnp.float32).at[:, :, :NI].set(
        wy1.reshape(B, NGRP, NI))

    T2 = U.reshape(B * H, W * C)
    out = _sc_sample(T2, scal, xm0, xm1, wxa0, wxa1, yr0, yr1, wya0, wya1,
                     B, H, W, C)
    return out.reshape(B, C, oh, ow)
